# baseline (speedup 1.0000x reference)
import jax
import jax.numpy as jnp
from jax import lax
from jax.experimental import pallas as pl
from jax.experimental.pallas import tpu as pltpu
from jax.experimental.pallas import tpu_sc as plsc

VOCAB = 1000000
BATCH = 4096
SEQ = 200
EMBED_DIM = 64

_INFO = plsc.get_sparse_core_info()
NC = _INFO.num_cores
NS = _INFO.num_subcores
NW = NC * NS
LANES = 16
VECS = EMBED_DIM // LANES

VTILE = 128
FULL_TILES = VOCAB // VTILE
TAIL = VOCAB - FULL_TILES * VTILE
GTILES = 2
GV = GTILES * VTILE
NGROUPS = FULL_TILES // GTILES
GROUPS_PER_W = -(-NGROUPS // NW)
GBLOCK_F = GV * EMBED_DIM
SLAB = 8


def _format_body(tt_hbm, tail_hbm, out_hbm,
                 in_v0, in_v1, out_v0, out_v1, in_sems, out_sems):
    wid = lax.axis_index("s") * NC + lax.axis_index("c")
    g0 = jnp.minimum(GROUPS_PER_W * wid, NGROUPS - GROUPS_PER_W)
    in_bufs = (in_v0, in_v1)
    out_bufs = (out_v0, out_v1)

    def start_group(g, buf):
        for r in range(EMBED_DIM // SLAB):
            pltpu.async_copy(
                tt_hbm.at[pl.ds(SLAB * r, SLAB), pl.ds(g * GV, GV)],
                in_bufs[buf].at[pl.ds(SLAB * r, SLAB)],
                in_sems.at[buf],
            )

    def wait_in(buf):
        pltpu.make_async_copy(
            tt_hbm.at[pl.ds(0, EMBED_DIM), pl.ds(0, GV)],
            in_bufs[buf],
            in_sems.at[buf],
        ).wait()

    def wait_out(buf):
        pltpu.make_async_copy(
            out_bufs[buf],
            out_hbm.at[pl.ds(0, GBLOCK_F)],
            out_sems.at[buf],
        ).wait()

    bases = [
        lax.iota(jnp.int32, LANES) * EMBED_DIM + (j * LANES * EMBED_DIM)
        for j in range(GV // LANES)
    ]

    def transpose_block(buf):
        iv, ov = in_bufs[buf], out_bufs[buf]

        def row_body(d, _):
            vecs = [iv[d, pl.ds(j * LANES, LANES)] for j in range(GV // LANES)]
            for j, vec in enumerate(vecs):
                plsc.store_scatter(ov, [bases[j] + d], vec)
            return 0

        lax.fori_loop(0, EMBED_DIM, row_body, 0, unroll=2)

    start_group(g0, 0)

    def loop_body(i, _):
        g = g0 + i
        for buf in range(2):
            @pl.when(lax.rem(i, 2) == buf)
            def _():
                wait_in(buf)

                @pl.when(i + 1 < GROUPS_PER_W)
                def _():
                    start_group(g + 1, 1 - buf)

                @pl.when(i >= 2)
                def _():
                    wait_out(buf)

                transpose_block(buf)
                pltpu.async_copy(
                    out_bufs[buf],
                    out_hbm.at[pl.ds(g * GBLOCK_F, GBLOCK_F)],
                    out_sems.at[buf],
                )
        return 0

    lax.fori_loop(0, GROUPS_PER_W, loop_body, 0)
    wait_out(0)
    wait_out(1)

    @pl.when(wid == NW - 1)
    def _():
        n_tail = TAIL * EMBED_DIM
        pltpu.async_copy(
            tail_hbm, out_v0.at[pl.ds(0, n_tail)], in_sems.at[0])
        pltpu.make_async_copy(
            tail_hbm, out_v0.at[pl.ds(0, n_tail)], in_sems.at[0]).wait()
        pltpu.async_copy(
            out_v0.at[pl.ds(0, n_tail)],
            out_hbm.at[pl.ds(FULL_TILES * VTILE * EMBED_DIM, n_tail)],
            out_sems.at[0],
        )
        pltpu.make_async_copy(
            out_v0.at[pl.ds(0, n_tail)],
            out_hbm.at[pl.ds(0, n_tail)],
            out_sems.at[0],
        ).wait()


_format = pl.kernel(
    _format_body,
    out_type=jax.ShapeDtypeStruct((VOCAB * EMBED_DIM,), jnp.float32),
    mesh=plsc.VectorSubcoreMesh(core_axis_name="c", subcore_axis_name="s"),
    scratch_types=[
        pltpu.VMEM((EMBED_DIM, GV), jnp.float32),
        pltpu.VMEM((EMBED_DIM, GV), jnp.float32),
        pltpu.VMEM((GBLOCK_F,), jnp.float32),
        pltpu.VMEM((GBLOCK_F,), jnp.float32),
        pltpu.SemaphoreType.DMA((2,)),
        pltpu.SemaphoreType.DMA((2,)),
    ],
    compiler_params=pltpu.CompilerParams(
        use_tc_tiling_on_sc=True, needs_layout_passes=False),
)

ROWS_PER_W = BATCH // NW
CHUNK = 100
CHUNKS_PER_ROW = SEQ // CHUNK
CHUNKS_PER_W = ROWS_PER_W * CHUNKS_PER_ROW
RING_ROWS = 4
NBUF = RING_ROWS * CHUNKS_PER_ROW


def _encoder_body(ids_hbm, table_hbm, out_hbm, idx_v, rows_v, out_v, sems):
    wid = lax.axis_index("s") * NC + lax.axis_index("c")

    pltpu.sync_copy(ids_hbm.at[wid], idx_v)

    inv_seq = jnp.float32(1.0 / SEQ)

    def start_row(b, bufs):
        for h in range(CHUNKS_PER_ROW):
            pltpu.async_copy(
                table_hbm.at[idx_v.at[b * CHUNKS_PER_ROW + h]],
                rows_v.at[bufs[h]],
                sems.at[bufs[h]],
            )

    for r in range(RING_ROWS):
        start_row(r, (2 * r, 2 * r + 1))

    def outer_body(o, _):
        base = o * RING_ROWS
        for bb in range(RING_ROWS):
            b = base + bb
            bufs = (2 * bb, 2 * bb + 1)
            acc = tuple(jnp.zeros((LANES,), jnp.float32) for _ in range(VECS))
            for h in range(CHUNKS_PER_ROW):
                buf = bufs[h]
                pltpu.make_async_copy(
                    table_hbm.at[idx_v.at[0]], rows_v.at[buf], sems.at[buf]
                ).wait()

                def acc_body(r, carry, buf=buf):
                    return tuple(
                        carry[k] + rows_v[buf, r, pl.ds(k * LANES, LANES)]
                        for k in range(VECS)
                    )

                acc = lax.fori_loop(0, CHUNK, acc_body, acc, unroll=2)
            for k in range(VECS):
                out_v[b, pl.ds(k * LANES, LANES)] = acc[k] * inv_seq

            @pl.when(b + RING_ROWS < ROWS_PER_W)
            def _():
                start_row(b + RING_ROWS, bufs)

        return 0

    lax.fori_loop(0, ROWS_PER_W // RING_ROWS, outer_body, 0)

    pltpu.sync_copy(out_v, out_hbm.at[pl.ds(wid * ROWS_PER_W, ROWS_PER_W)])


_encoder = pl.kernel(
    _encoder_body,
    out_type=jax.ShapeDtypeStruct((BATCH, EMBED_DIM), jnp.float32),
    mesh=plsc.VectorSubcoreMesh(core_axis_name="c", subcore_axis_name="s"),
    scratch_types=[
        pltpu.VMEM((CHUNKS_PER_W, CHUNK), jnp.int32),
        pltpu.VMEM((NBUF, CHUNK, EMBED_DIM), jnp.float32),
        pltpu.VMEM((ROWS_PER_W, EMBED_DIM), jnp.float32),
        pltpu.SemaphoreType.DMA((NBUF,)),
    ],
    compiler_params=pltpu.CompilerParams(use_tc_tiling_on_sc=False),
)


@jax.jit
def kernel(text_ids, table):
    ids = text_ids.astype(jnp.int32).reshape(NW, CHUNKS_PER_W, CHUNK)
    tail = lax.slice(table, (FULL_TILES * VTILE, 0), (VOCAB, EMBED_DIM))
    tlin = _format(table.T, tail.reshape(TAIL * EMBED_DIM))
    tab = tlin.reshape(VOCAB, EMBED_DIM)
    return _encoder(ids, tab)


# speedup vs baseline: 1.9337x; 1.9337x over previous
import jax
import jax.numpy as jnp
from jax import lax
from jax.experimental import pallas as pl
from jax.experimental.pallas import tpu as pltpu
from jax.experimental.pallas import tpu_sc as plsc

BATCH = 4096
SEQ = 200
EMBED_DIM = 64

_INFO = plsc.get_sparse_core_info()
NC = _INFO.num_cores
NS = _INFO.num_subcores
NW = NC * NS
ROWS_PER_W = BATCH // NW
CHUNK = 100
CHUNKS_PER_ROW = SEQ // CHUNK
CHUNKS_PER_W = ROWS_PER_W * CHUNKS_PER_ROW
LANES = 16
VECS = EMBED_DIM // LANES
RING_ROWS = 4
NBUF = RING_ROWS * CHUNKS_PER_ROW


def _encoder_body(ids_hbm, table_hbm, out_hbm, idx_v, rows_v, out_v, sems):
    wid = lax.axis_index("s") * NC + lax.axis_index("c")

    pltpu.sync_copy(ids_hbm.at[wid], idx_v)

    inv_seq = jnp.float32(1.0 / SEQ)

    def start_row(b, bufs):
        for h in range(CHUNKS_PER_ROW):
            pltpu.async_copy(
                table_hbm.at[idx_v.at[b * CHUNKS_PER_ROW + h]],
                rows_v.at[bufs[h]],
                sems.at[bufs[h]],
            )

    for r in range(RING_ROWS):
        start_row(r, (2 * r, 2 * r + 1))

    def outer_body(o, _):
        base = o * RING_ROWS
        for bb in range(RING_ROWS):
            b = base + bb
            bufs = (2 * bb, 2 * bb + 1)
            acc = tuple(jnp.zeros((LANES,), jnp.float32) for _ in range(VECS))
            for h in range(CHUNKS_PER_ROW):
                buf = bufs[h]
                pltpu.make_async_copy(
                    table_hbm.at[idx_v.at[0]], rows_v.at[buf], sems.at[buf]
                ).wait()

                def acc_body(r, carry, buf=buf):
                    return tuple(
                        carry[k] + rows_v[buf, r, pl.ds(k * LANES, LANES)]
                        for k in range(VECS)
                    )

                acc = lax.fori_loop(0, CHUNK, acc_body, acc, unroll=2)
            for k in range(VECS):
                out_v[b, pl.ds(k * LANES, LANES)] = acc[k] * inv_seq

            @pl.when(b + RING_ROWS < ROWS_PER_W)
            def _():
                start_row(b + RING_ROWS, bufs)

        return 0

    lax.fori_loop(0, ROWS_PER_W // RING_ROWS, outer_body, 0)

    pltpu.sync_copy(out_v, out_hbm.at[pl.ds(wid * ROWS_PER_W, ROWS_PER_W)])


_encoder = pl.kernel(
    _encoder_body,
    out_type=jax.ShapeDtypeStruct((BATCH, EMBED_DIM), jnp.float32),
    mesh=plsc.VectorSubcoreMesh(core_axis_name="c", subcore_axis_name="s"),
    scratch_types=[
        pltpu.VMEM((CHUNKS_PER_W, CHUNK), jnp.int32),
        pltpu.VMEM((NBUF, CHUNK, EMBED_DIM), jnp.float32),
        pltpu.VMEM((ROWS_PER_W, EMBED_DIM), jnp.float32),
        pltpu.SemaphoreType.DMA((NBUF,)),
    ],
    compiler_params=pltpu.CompilerParams(use_tc_tiling_on_sc=False),
)


@jax.jit
def kernel(text_ids, table):
    ids = text_ids.astype(jnp.int32).reshape(NW, CHUNKS_PER_W, CHUNK)
    return _encoder(ids, table)
